# baseline (speedup 1.0000x reference)
import functools

import jax
import jax.numpy as jnp
from jax import lax
from jax.experimental import pallas as pl
from jax.experimental.pallas import tpu as pltpu
from jax.experimental.pallas import tpu_sc as plsc

PS = 64
K = 64
B = 8
G = 16
NP = G * G
ROWS_PER_PATCH = PS
ROW_W = PS * 3
N_TABLE_ROWS = B * 1024 * G
N_OUT_ROWS = B * K * ROWS_PER_PATCH


def _pool_body(hm_ref, out_ref):
    g = pl.program_id(1)
    x = hm_ref[0]
    colsum = jnp.sum(x, axis=0, keepdims=True)
    r = lax.broadcasted_iota(jnp.int32, (1024, G), 0)
    c = lax.broadcasted_iota(jnp.int32, (1024, G), 1)
    grp = ((r // PS) == c).astype(jnp.float32)
    row = lax.dot_general(colsum, grp, (((1,), (0,)), ((), ())),
                          precision=lax.Precision.HIGHEST,
                          preferred_element_type=jnp.float32)
    out_ref[0, pl.ds(g, 1), :] = row * (1.0 / (PS * PS))


def _pool(hm):
    return pl.pallas_call(
        _pool_body,
        grid=(B, G),
        in_specs=[pl.BlockSpec((1, PS, 1024), lambda b, g: (b, g, 0))],
        out_specs=pl.BlockSpec((1, G, G), lambda b, g: (b, 0, 0)),
        out_shape=jax.ShapeDtypeStruct((B, G, G), jnp.float32),
    )(hm)


def _col_of(row_vec, n):
    i = lax.broadcasted_iota(jnp.int32, (n, n), 0)
    j = lax.broadcasted_iota(jnp.int32, (n, n), 1)
    diag = (i == j).astype(row_vec.dtype)
    return jnp.sum(diag * row_vec, axis=1, keepdims=True)


def _topk_body(avg_ref, out_ref):
    b = pl.program_id(0)
    val = avg_ref[0]
    val_col = _col_of(val, NP)
    i_lane = lax.broadcasted_iota(jnp.int32, (NP, NP), 1)
    j_sub = lax.broadcasted_iota(jnp.int32, (NP, NP), 0)
    beats = (val_col > val) | ((val_col == val) & (j_sub < i_lane))
    rank = jnp.sum(beats.astype(jnp.float32), axis=0, keepdims=True)
    rank_col = _col_of(rank, NP)
    p = lax.broadcasted_iota(jnp.int32, (1, K), 1).astype(jnp.float32)
    onehot = (rank_col == p).astype(jnp.float32)
    i_col = lax.broadcasted_iota(jnp.int32, (NP, 1), 0).astype(jnp.float32)
    g_row = jnp.sum(onehot * i_col, axis=0, keepdims=True)
    g_col = _col_of(g_row, K).astype(jnp.int32)
    gh = g_col >> 4
    gw = g_col & 15
    base = b * (1024 * G) + gh * (PS * G) + gw
    step = lax.broadcasted_iota(jnp.int32, (1, ROWS_PER_PATCH), 1) * G
    out_ref[:] = base + step


def _topk_rows(avg):
    return pl.pallas_call(
        _topk_body,
        grid=(B,),
        in_specs=[pl.BlockSpec((1, 1, NP), lambda b: (b, 0, 0))],
        out_specs=pl.BlockSpec((K, ROWS_PER_PATCH), lambda b: (b, 0)),
        out_shape=jax.ShapeDtypeStruct((B * K, ROWS_PER_PATCH), jnp.int32),
    )(avg)


CHUNK = 128


def _gather_body(num_cores, rows_per_worker, table_hbm, idx_hbm, out_hbm,
                 idx_v, buf0, buf1, sem0, sem1):
    n_chunks = rows_per_worker // CHUNK
    wid = lax.axis_index("s") * num_cores + lax.axis_index("c")
    pltpu.sync_copy(idx_hbm.at[wid], idx_v)
    bufs = (buf0, buf1)
    sems = (sem0, sem1)
    cps = [None, None]
    cps[0] = pltpu.async_copy(table_hbm.at[idx_v.at[0]], bufs[0], sems[0])
    for c in range(n_chunks):
        nxt = (c + 1) % 2
        if c + 1 < n_chunks:
            cps[nxt] = pltpu.async_copy(
                table_hbm.at[idx_v.at[c + 1]], bufs[nxt], sems[nxt])
        cps[c % 2].wait()
        pltpu.sync_copy(
            bufs[c % 2],
            out_hbm.at[pl.ds(wid * rows_per_worker + c * CHUNK, CHUNK)])


def _gather(table, idx_rows):
    info = plsc.get_sparse_core_info()
    nw = info.num_cores * info.num_subcores
    rows_per_worker = N_OUT_ROWS // nw
    idx3 = idx_rows.reshape(nw, rows_per_worker // CHUNK, CHUNK)
    mesh = plsc.VectorSubcoreMesh(core_axis_name="c", subcore_axis_name="s")
    body = functools.partial(_gather_body, info.num_cores, rows_per_worker)
    return pl.kernel(
        body,
        out_type=jax.ShapeDtypeStruct((N_OUT_ROWS, ROW_W), jnp.float32),
        mesh=mesh,
        compiler_params=pltpu.CompilerParams(use_tc_tiling_on_sc=False),
        scratch_types=[
            pltpu.VMEM((rows_per_worker // CHUNK, CHUNK), jnp.int32),
            pltpu.VMEM((CHUNK, ROW_W), jnp.float32),
            pltpu.VMEM((CHUNK, ROW_W), jnp.float32),
            pltpu.SemaphoreType.DMA,
            pltpu.SemaphoreType.DMA,
        ],
    )(table, idx3)


def _tc_gather_body(idx_ref, img_lo_ref, img_hi_ref, out_ref):
    n = pl.program_id(0)
    parity = idx_ref[n] & 1
    xl = img_lo_ref[...]
    xh = img_hi_ref[...]
    out_ref[0, :, :PS // 2] = jnp.where(parity == 0, xl[:, :, :PS],
                                        xl[:, :, PS:])
    out_ref[0, :, PS // 2:] = jnp.where(parity == 0, xh[:, :, :PS],
                                        xh[:, :, PS:])


def _tc_gather(planar, idx):
    def img_map_lo(n, idx_ref):
        g = idx_ref[n]
        return (n // K, (g >> 4) * 2, (g & 15) >> 1)

    def img_map_hi(n, idx_ref):
        g = idx_ref[n]
        return (n // K, (g >> 4) * 2 + 1, (g & 15) >> 1)

    grid_spec = pltpu.PrefetchScalarGridSpec(
        num_scalar_prefetch=1,
        grid=(B * K,),
        in_specs=[pl.BlockSpec((3, PS // 2, 2 * PS), img_map_lo),
                  pl.BlockSpec((3, PS // 2, 2 * PS), img_map_hi)],
        out_specs=pl.BlockSpec((1, 3, PS, PS),
                               lambda n, idx_ref: (n, 0, 0, 0)),
    )
    return pl.pallas_call(
        _tc_gather_body,
        grid_spec=grid_spec,
        out_shape=jax.ShapeDtypeStruct((B * K, 3, PS, PS), jnp.float32),
    )(idx, planar, planar)


def kernel(heatmap, image):
    hm = heatmap.reshape(B, 1024, 1024)
    avg = _pool(hm).reshape(B, 1, NP)
    rows = _topk_rows(avg)
    planar = image.transpose(0, 3, 1, 2).reshape(B * 3, 1024, 1024)
    z = _tc_gather(planar, _rows_to_g(rows))
    return z.transpose(0, 2, 3, 1)


def _rows_to_g(rows):
    row0 = rows[:, 0] % (1024 * G)
    return ((row0 >> 10) << 4) | (row0 & 15)


# speedup vs baseline: 1.9546x; 1.9546x over previous
import functools

import jax
import jax.numpy as jnp
from jax import lax
from jax.experimental import pallas as pl
from jax.experimental.pallas import tpu as pltpu
from jax.experimental.pallas import tpu_sc as plsc

PS = 64
K = 64
B = 8
G = 16
NP = G * G
ROWS_PER_PATCH = PS
ROW_W = PS * 3
N_TABLE_ROWS = B * 1024 * G
N_OUT_ROWS = B * K * ROWS_PER_PATCH


def _pool_body(hm_ref, out_ref):
    g = pl.program_id(1)
    x = hm_ref[0]
    colsum = jnp.sum(x, axis=0, keepdims=True)
    r = lax.broadcasted_iota(jnp.int32, (1024, G), 0)
    c = lax.broadcasted_iota(jnp.int32, (1024, G), 1)
    grp = ((r // PS) == c).astype(jnp.float32)
    row = lax.dot_general(colsum, grp, (((1,), (0,)), ((), ())),
                          precision=lax.Precision.HIGHEST,
                          preferred_element_type=jnp.float32)
    out_ref[0, pl.ds(g, 1), :] = row * (1.0 / (PS * PS))


def _pool(hm):
    return pl.pallas_call(
        _pool_body,
        grid=(B, G),
        in_specs=[pl.BlockSpec((1, PS, 1024), lambda b, g: (b, g, 0))],
        out_specs=pl.BlockSpec((1, G, G), lambda b, g: (b, 0, 0)),
        out_shape=jax.ShapeDtypeStruct((B, G, G), jnp.float32),
    )(hm)


def _col_of(row_vec, n):
    i = lax.broadcasted_iota(jnp.int32, (n, n), 0)
    j = lax.broadcasted_iota(jnp.int32, (n, n), 1)
    diag = (i == j).astype(row_vec.dtype)
    return jnp.sum(diag * row_vec, axis=1, keepdims=True)


def _topk_body(avg_ref, out_ref):
    b = pl.program_id(0)
    val = avg_ref[0]
    val_col = _col_of(val, NP)
    i_lane = lax.broadcasted_iota(jnp.int32, (NP, NP), 1)
    j_sub = lax.broadcasted_iota(jnp.int32, (NP, NP), 0)
    beats = (val_col > val) | ((val_col == val) & (j_sub < i_lane))
    rank = jnp.sum(beats.astype(jnp.float32), axis=0, keepdims=True)
    rank_col = _col_of(rank, NP)
    p = lax.broadcasted_iota(jnp.int32, (1, K), 1).astype(jnp.float32)
    onehot = (rank_col == p).astype(jnp.float32)
    i_col = lax.broadcasted_iota(jnp.int32, (NP, 1), 0).astype(jnp.float32)
    g_row = jnp.sum(onehot * i_col, axis=0, keepdims=True)
    g_col = _col_of(g_row, K).astype(jnp.int32)
    gh = g_col >> 4
    gw = g_col & 15
    base = b * (1024 * G) + gh * (PS * G) + gw
    step = lax.broadcasted_iota(jnp.int32, (1, ROWS_PER_PATCH), 1) * G
    out_ref[:] = base + step


def _topk_rows(avg):
    return pl.pallas_call(
        _topk_body,
        grid=(B,),
        in_specs=[pl.BlockSpec((1, 1, NP), lambda b: (b, 0, 0))],
        out_specs=pl.BlockSpec((K, ROWS_PER_PATCH), lambda b: (b, 0)),
        out_shape=jax.ShapeDtypeStruct((B * K, ROWS_PER_PATCH), jnp.int32),
    )(avg)


CHUNK = 128


def _gather_body(num_cores, rows_per_worker, table_hbm, idx_hbm, out_hbm,
                 idx_v, buf0, buf1, sem0, sem1):
    n_chunks = rows_per_worker // CHUNK
    wid = lax.axis_index("s") * num_cores + lax.axis_index("c")
    pltpu.sync_copy(idx_hbm.at[wid], idx_v)
    bufs = (buf0, buf1)
    sems = (sem0, sem1)
    cps = [None, None]
    cps[0] = pltpu.async_copy(table_hbm.at[idx_v.at[0]], bufs[0], sems[0])
    for c in range(n_chunks):
        nxt = (c + 1) % 2
        if c + 1 < n_chunks:
            cps[nxt] = pltpu.async_copy(
                table_hbm.at[idx_v.at[c + 1]], bufs[nxt], sems[nxt])
        cps[c % 2].wait()
        pltpu.sync_copy(
            bufs[c % 2],
            out_hbm.at[pl.ds(wid * rows_per_worker + c * CHUNK, CHUNK)])


def _gather(table, idx_rows):
    info = plsc.get_sparse_core_info()
    nw = info.num_cores * info.num_subcores
    rows_per_worker = N_OUT_ROWS // nw
    idx3 = idx_rows.reshape(nw, rows_per_worker // CHUNK, CHUNK)
    mesh = plsc.VectorSubcoreMesh(core_axis_name="c", subcore_axis_name="s")
    body = functools.partial(_gather_body, info.num_cores, rows_per_worker)
    return pl.kernel(
        body,
        out_type=jax.ShapeDtypeStruct((N_OUT_ROWS, ROW_W), jnp.float32),
        mesh=mesh,
        compiler_params=pltpu.CompilerParams(use_tc_tiling_on_sc=False),
        scratch_types=[
            pltpu.VMEM((rows_per_worker // CHUNK, CHUNK), jnp.int32),
            pltpu.VMEM((CHUNK, ROW_W), jnp.float32),
            pltpu.VMEM((CHUNK, ROW_W), jnp.float32),
            pltpu.SemaphoreType.DMA,
            pltpu.SemaphoreType.DMA,
        ],
    )(table, idx3)


PPS = 8


def _tc_gather_body(idx_ref, *refs):
    n = pl.program_id(0)
    img_refs, out_ref = refs[:PPS], refs[PPS]
    for j in range(PPS):
        parity = idx_ref[n * PPS + j] & 1
        x = img_refs[j][...]
        out_ref[j] = jnp.where(parity == 0, x[:, :, :PS], x[:, :, PS:])


def _tc_gather(planar, idx):
    def img_map(j):
        def m(n, idx_ref):
            g = idx_ref[n * PPS + j]
            return ((n * PPS + j) // K, g >> 4, (g & 15) >> 1)
        return m

    grid_spec = pltpu.PrefetchScalarGridSpec(
        num_scalar_prefetch=1,
        grid=(B * K // PPS,),
        in_specs=[pl.BlockSpec((3, PS, 2 * PS), img_map(j))
                  for j in range(PPS)],
        out_specs=pl.BlockSpec((PPS, 3, PS, PS),
                               lambda n, idx_ref: (n, 0, 0, 0)),
    )
    return pl.pallas_call(
        _tc_gather_body,
        grid_spec=grid_spec,
        out_shape=jax.ShapeDtypeStruct((B * K, 3, PS, PS), jnp.float32),
    )(idx, *([planar] * PPS))


def kernel(heatmap, image):
    hm = heatmap.reshape(B, 1024, 1024)
    avg = _pool(hm).reshape(B, 1, NP)
    rows = _topk_rows(avg)
    planar = image.transpose(0, 3, 1, 2).reshape(B * 3, 1024, 1024)
    z = _tc_gather(planar, _rows_to_g(rows))
    return z.transpose(0, 2, 3, 1)


def _rows_to_g(rows):
    row0 = rows[:, 0] % (1024 * G)
    return ((row0 >> 10) << 4) | (row0 & 15)


# speedup vs baseline: 2.1218x; 1.0856x over previous
import functools

import jax
import jax.numpy as jnp
from jax import lax
from jax.experimental import pallas as pl
from jax.experimental.pallas import tpu as pltpu
from jax.experimental.pallas import tpu_sc as plsc

PS = 64
K = 64
B = 8
G = 16
NP = G * G
ROWS_PER_PATCH = PS
ROW_W = PS * 3
N_TABLE_ROWS = B * 1024 * G
N_OUT_ROWS = B * K * ROWS_PER_PATCH


def _pool_body(hm_ref, out_ref):
    g = pl.program_id(1)
    x = hm_ref[0]
    colsum = jnp.sum(x, axis=0, keepdims=True)
    r = lax.broadcasted_iota(jnp.int32, (1024, G), 0)
    c = lax.broadcasted_iota(jnp.int32, (1024, G), 1)
    grp = ((r // PS) == c).astype(jnp.float32)
    row = lax.dot_general(colsum, grp, (((1,), (0,)), ((), ())),
                          precision=lax.Precision.HIGHEST,
                          preferred_element_type=jnp.float32)
    out_ref[0, pl.ds(g, 1), :] = row * (1.0 / (PS * PS))


def _pool(hm):
    return pl.pallas_call(
        _pool_body,
        grid=(B, G),
        in_specs=[pl.BlockSpec((1, PS, 1024), lambda b, g: (b, g, 0))],
        out_specs=pl.BlockSpec((1, G, G), lambda b, g: (b, 0, 0)),
        out_shape=jax.ShapeDtypeStruct((B, G, G), jnp.float32),
    )(hm)


def _col_of(row_vec, n):
    i = lax.broadcasted_iota(jnp.int32, (n, n), 0)
    j = lax.broadcasted_iota(jnp.int32, (n, n), 1)
    diag = (i == j).astype(row_vec.dtype)
    return jnp.sum(diag * row_vec, axis=1, keepdims=True)


def _topk_body(avg_ref, out_ref):
    b = pl.program_id(0)
    val = avg_ref[0]
    val_col = _col_of(val, NP)
    i_lane = lax.broadcasted_iota(jnp.int32, (NP, NP), 1)
    j_sub = lax.broadcasted_iota(jnp.int32, (NP, NP), 0)
    beats = (val_col > val) | ((val_col == val) & (j_sub < i_lane))
    rank = jnp.sum(beats.astype(jnp.float32), axis=0, keepdims=True)
    rank_col = _col_of(rank, NP)
    p = lax.broadcasted_iota(jnp.int32, (1, K), 1).astype(jnp.float32)
    onehot = (rank_col == p).astype(jnp.float32)
    i_col = lax.broadcasted_iota(jnp.int32, (NP, 1), 0).astype(jnp.float32)
    g_row = jnp.sum(onehot * i_col, axis=0, keepdims=True)
    g_col = _col_of(g_row, K).astype(jnp.int32)
    gh = g_col >> 4
    gw = g_col & 15
    base = b * (1024 * G) + gh * (PS * G) + gw
    step = lax.broadcasted_iota(jnp.int32, (1, ROWS_PER_PATCH), 1) * G
    out_ref[:] = base + step


def _topk_rows(avg):
    return pl.pallas_call(
        _topk_body,
        grid=(B,),
        in_specs=[pl.BlockSpec((1, 1, NP), lambda b: (b, 0, 0))],
        out_specs=pl.BlockSpec((K, ROWS_PER_PATCH), lambda b: (b, 0)),
        out_shape=jax.ShapeDtypeStruct((B * K, ROWS_PER_PATCH), jnp.int32),
    )(avg)


CHUNK = 128


def _gather_body(num_cores, rows_per_worker, table_hbm, idx_hbm, out_hbm,
                 idx_v, buf0, buf1, sem0, sem1):
    n_chunks = rows_per_worker // CHUNK
    wid = lax.axis_index("s") * num_cores + lax.axis_index("c")
    pltpu.sync_copy(idx_hbm.at[wid], idx_v)
    bufs = (buf0, buf1)
    sems = (sem0, sem1)
    cps = [None, None]
    cps[0] = pltpu.async_copy(table_hbm.at[idx_v.at[0]], bufs[0], sems[0])
    for c in range(n_chunks):
        nxt = (c + 1) % 2
        if c + 1 < n_chunks:
            cps[nxt] = pltpu.async_copy(
                table_hbm.at[idx_v.at[c + 1]], bufs[nxt], sems[nxt])
        cps[c % 2].wait()
        pltpu.sync_copy(
            bufs[c % 2],
            out_hbm.at[pl.ds(wid * rows_per_worker + c * CHUNK, CHUNK)])


def _gather(table, idx_rows):
    info = plsc.get_sparse_core_info()
    nw = info.num_cores * info.num_subcores
    rows_per_worker = N_OUT_ROWS // nw
    idx3 = idx_rows.reshape(nw, rows_per_worker // CHUNK, CHUNK)
    mesh = plsc.VectorSubcoreMesh(core_axis_name="c", subcore_axis_name="s")
    body = functools.partial(_gather_body, info.num_cores, rows_per_worker)
    return pl.kernel(
        body,
        out_type=jax.ShapeDtypeStruct((N_OUT_ROWS, ROW_W), jnp.float32),
        mesh=mesh,
        compiler_params=pltpu.CompilerParams(use_tc_tiling_on_sc=False),
        scratch_types=[
            pltpu.VMEM((rows_per_worker // CHUNK, CHUNK), jnp.int32),
            pltpu.VMEM((CHUNK, ROW_W), jnp.float32),
            pltpu.VMEM((CHUNK, ROW_W), jnp.float32),
            pltpu.SemaphoreType.DMA,
            pltpu.SemaphoreType.DMA,
        ],
    )(table, idx3)


PPS = 16


def _tc_gather_body(idx_ref, *refs):
    n = pl.program_id(0)
    img_refs, out_ref = refs[:PPS], refs[PPS]
    for j in range(PPS):
        parity = idx_ref[n * PPS + j] & 1
        x = img_refs[j][...]
        out_ref[j] = jnp.where(parity == 0, x[:, :, :PS], x[:, :, PS:])


def _tc_gather(planar, idx):
    def img_map(j):
        def m(n, idx_ref):
            g = idx_ref[n * PPS + j]
            return ((n * PPS + j) // K, g >> 4, (g & 15) >> 1)
        return m

    grid_spec = pltpu.PrefetchScalarGridSpec(
        num_scalar_prefetch=1,
        grid=(B * K // PPS,),
        in_specs=[pl.BlockSpec((3, PS, 2 * PS), img_map(j))
                  for j in range(PPS)],
        out_specs=pl.BlockSpec((PPS, 3, PS, PS),
                               lambda n, idx_ref: (n, 0, 0, 0)),
    )
    return pl.pallas_call(
        _tc_gather_body,
        grid_spec=grid_spec,
        out_shape=jax.ShapeDtypeStruct((B * K, 3, PS, PS), jnp.float32),
    )(idx, *([planar] * PPS))


def kernel(heatmap, image):
    hm = heatmap.reshape(B, 1024, 1024)
    avg = _pool(hm).reshape(B, 1, NP)
    rows = _topk_rows(avg)
    planar = image.transpose(0, 3, 1, 2).reshape(B * 3, 1024, 1024)
    z = _tc_gather(planar, _rows_to_g(rows))
    return z.transpose(0, 2, 3, 1)


def _rows_to_g(rows):
    row0 = rows[:, 0] % (1024 * G)
    return ((row0 >> 10) << 4) | (row0 & 15)


# speedup vs baseline: 3.6296x; 1.7106x over previous
import functools

import jax
import jax.numpy as jnp
from jax import lax
from jax.experimental import pallas as pl
from jax.experimental.pallas import tpu as pltpu
from jax.experimental.pallas import tpu_sc as plsc

PS = 64
K = 64
B = 8
G = 16
NP = G * G
ROWS_PER_PATCH = PS
ROW_W = PS * 3
N_TABLE_ROWS = B * 1024 * G
N_OUT_ROWS = B * K * ROWS_PER_PATCH


def _pool_body(hm_ref, out_ref):
    g = pl.program_id(1)
    x = hm_ref[0]
    colsum = jnp.sum(x, axis=0, keepdims=True)
    r = lax.broadcasted_iota(jnp.int32, (1024, G), 0)
    c = lax.broadcasted_iota(jnp.int32, (1024, G), 1)
    grp = ((r // PS) == c).astype(jnp.float32)
    row = lax.dot_general(colsum, grp, (((1,), (0,)), ((), ())),
                          precision=lax.Precision.HIGHEST,
                          preferred_element_type=jnp.float32)
    out_ref[0, pl.ds(g, 1), :] = row * (1.0 / (PS * PS))


def _pool(hm):
    return pl.pallas_call(
        _pool_body,
        grid=(B, G),
        in_specs=[pl.BlockSpec((1, PS, 1024), lambda b, g: (b, g, 0))],
        out_specs=pl.BlockSpec((1, G, G), lambda b, g: (b, 0, 0)),
        out_shape=jax.ShapeDtypeStruct((B, G, G), jnp.float32),
    )(hm)


def _col_of(row_vec, n):
    i = lax.broadcasted_iota(jnp.int32, (n, n), 0)
    j = lax.broadcasted_iota(jnp.int32, (n, n), 1)
    diag = (i == j).astype(row_vec.dtype)
    return jnp.sum(diag * row_vec, axis=1, keepdims=True)


def _topk_body(avg_ref, out_ref):
    b = pl.program_id(0)
    val = avg_ref[0]
    val_col = _col_of(val, NP)
    i_lane = lax.broadcasted_iota(jnp.int32, (NP, NP), 1)
    j_sub = lax.broadcasted_iota(jnp.int32, (NP, NP), 0)
    beats = (val_col > val) | ((val_col == val) & (j_sub < i_lane))
    rank = jnp.sum(beats.astype(jnp.float32), axis=0, keepdims=True)
    rank_col = _col_of(rank, NP)
    p = lax.broadcasted_iota(jnp.int32, (1, K), 1).astype(jnp.float32)
    onehot = (rank_col == p).astype(jnp.float32)
    i_col = lax.broadcasted_iota(jnp.int32, (NP, 1), 0).astype(jnp.float32)
    g_row = jnp.sum(onehot * i_col, axis=0, keepdims=True)
    g_col = _col_of(g_row, K).astype(jnp.int32)
    gh = g_col >> 4
    gw = g_col & 15
    base = b * (1024 * G) + gh * (PS * G) + gw
    step = lax.broadcasted_iota(jnp.int32, (1, ROWS_PER_PATCH), 1) * G
    out_ref[:] = base + step


def _topk_rows(avg):
    return pl.pallas_call(
        _topk_body,
        grid=(B,),
        in_specs=[pl.BlockSpec((1, 1, NP), lambda b: (b, 0, 0))],
        out_specs=pl.BlockSpec((K, ROWS_PER_PATCH), lambda b: (b, 0)),
        out_shape=jax.ShapeDtypeStruct((B * K, ROWS_PER_PATCH), jnp.int32),
    )(avg)


def _sc_pool_topk_body(hm_ref, g_ref, stripe_v, avg16_v, avg2d_v, gsel_v,
                       sh_ref):
    c = lax.axis_index("c")
    s = lax.axis_index("s")
    b_local = s >> 2
    b = c * 4 + b_local

    if True:
        for k in range(4):
            gh = (s & 3) * 4 + k
            row0 = b * 1024 + gh * PS
            pltpu.sync_copy(hm_ref.at[pl.ds(row0, PS)], stripe_v)
            lane16 = lax.broadcasted_iota(jnp.int32, (16,), 0)
            avg_vec = jnp.zeros((16,), jnp.float32)
            for cb in range(4):
                def body(r, acc):
                    return tuple(
                        acc[v] + stripe_v[r, pl.ds(cb * 256 + v * 16, 16)]
                        for v in range(16))
                acc = lax.fori_loop(
                    0, PS, body,
                    tuple(jnp.zeros((16,), jnp.float32) for _ in range(16)))
                for q in range(4):
                    tot = acc[4 * q] + acc[4 * q + 1] + acc[4 * q + 2] \
                        + acc[4 * q + 3]
                    sc = jnp.broadcast_to(jnp.sum(tot), (16,))
                    avg_vec = jnp.where(lane16 == 4 * cb + q, sc, avg_vec)
            avg16_v[...] = avg_vec
            pltpu.sync_copy(avg16_v, sh_ref.at[b_local, gh])
        plsc.subcore_barrier()

        @pl.when(s < 4)
        def _topk():
            pltpu.sync_copy(sh_ref.at[s], avg2d_v)
            lane = lax.broadcasted_iota(jnp.int32, (16,), 0)

            vis = [avg2d_v[i, :] for i in range(16)]

            def body(j, ranks):
                vj = avg2d_v[j, :]
                new = list(ranks)
                for l in range(16):
                    val = jnp.broadcast_to(vj[l], (16,))
                    jl = j * 16 + l
                    for i in range(16):
                        ids_i = i * 16 + lane
                        beats = (val > vis[i]) | ((val == vis[i])
                                                  & (jl < ids_i))
                        new[i] = new[i] + jnp.where(beats, 1, 0)
                return tuple(new)
            ranks = lax.fori_loop(
                0, G, body,
                tuple(jnp.zeros((16,), jnp.int32) for _ in range(16)))
            for i in range(16):
                plsc.store_scatter(gsel_v, [ranks[i]], i * 16 + lane,
                                   mask=ranks[i] < K)
            cb2 = c * 4 + s
            pltpu.sync_copy(gsel_v, g_ref.at[pl.ds(cb2 * K, K)])


def _sc_pool_topk(hm_rows):
    mesh = plsc.VectorSubcoreMesh(core_axis_name="c", subcore_axis_name="s")
    return pl.kernel(
        _sc_pool_topk_body,
        out_type=jax.ShapeDtypeStruct((B * K,), jnp.int32),
        mesh=mesh,
        compiler_params=pltpu.CompilerParams(use_tc_tiling_on_sc=False,
                                             needs_layout_passes=False),
        scratch_types=[
            pltpu.VMEM((PS, 1024), jnp.float32),
            pltpu.VMEM((G,), jnp.float32),
            pltpu.VMEM((G, G), jnp.float32),
            pltpu.VMEM((K,), jnp.int32),
            pltpu.VMEM_SHARED((4, G, G), jnp.float32),
        ],
    )(hm_rows)


CHUNK = 128


def _gather_body(num_cores, rows_per_worker, table_hbm, idx_hbm, out_hbm,
                 idx_v, buf0, buf1, sem0, sem1):
    n_chunks = rows_per_worker // CHUNK
    wid = lax.axis_index("s") * num_cores + lax.axis_index("c")
    pltpu.sync_copy(idx_hbm.at[wid], idx_v)
    bufs = (buf0, buf1)
    sems = (sem0, sem1)
    cps = [None, None]
    cps[0] = pltpu.async_copy(table_hbm.at[idx_v.at[0]], bufs[0], sems[0])
    for c in range(n_chunks):
        nxt = (c + 1) % 2
        if c + 1 < n_chunks:
            cps[nxt] = pltpu.async_copy(
                table_hbm.at[idx_v.at[c + 1]], bufs[nxt], sems[nxt])
        cps[c % 2].wait()
        pltpu.sync_copy(
            bufs[c % 2],
            out_hbm.at[pl.ds(wid * rows_per_worker + c * CHUNK, CHUNK)])


def _gather(table, idx_rows):
    info = plsc.get_sparse_core_info()
    nw = info.num_cores * info.num_subcores
    rows_per_worker = N_OUT_ROWS // nw
    idx3 = idx_rows.reshape(nw, rows_per_worker // CHUNK, CHUNK)
    mesh = plsc.VectorSubcoreMesh(core_axis_name="c", subcore_axis_name="s")
    body = functools.partial(_gather_body, info.num_cores, rows_per_worker)
    return pl.kernel(
        body,
        out_type=jax.ShapeDtypeStruct((N_OUT_ROWS, ROW_W), jnp.float32),
        mesh=mesh,
        compiler_params=pltpu.CompilerParams(use_tc_tiling_on_sc=False),
        scratch_types=[
            pltpu.VMEM((rows_per_worker // CHUNK, CHUNK), jnp.int32),
            pltpu.VMEM((CHUNK, ROW_W), jnp.float32),
            pltpu.VMEM((CHUNK, ROW_W), jnp.float32),
            pltpu.SemaphoreType.DMA,
            pltpu.SemaphoreType.DMA,
        ],
    )(table, idx3)


PPS = 16


def _tc_gather_body(idx_ref, *refs):
    n = pl.program_id(0)
    img_refs, out_ref = refs[:PPS], refs[PPS]
    for j in range(PPS):
        parity = idx_ref[n * PPS + j] & 1
        x = img_refs[j][...]
        out_ref[j] = jnp.where(parity == 0, x[:, :, :PS], x[:, :, PS:])


def _tc_gather(planar, idx):
    def img_map(j):
        def m(n, idx_ref):
            g = idx_ref[n * PPS + j]
            return ((n * PPS + j) // K, g >> 4, (g & 15) >> 1)
        return m

    grid_spec = pltpu.PrefetchScalarGridSpec(
        num_scalar_prefetch=1,
        grid=(B * K // PPS,),
        in_specs=[pl.BlockSpec((3, PS, 2 * PS), img_map(j))
                  for j in range(PPS)],
        out_specs=pl.BlockSpec((PPS, 3, PS, PS),
                               lambda n, idx_ref: (n, 0, 0, 0)),
    )
    return pl.pallas_call(
        _tc_gather_body,
        grid_spec=grid_spec,
        out_shape=jax.ShapeDtypeStruct((B * K, 3, PS, PS), jnp.float32),
    )(idx, *([planar] * PPS))


def kernel(heatmap, image):
    hm_rows = heatmap.reshape(B * 1024, 1024)
    g = _sc_pool_topk(hm_rows)
    planar = image.transpose(0, 3, 1, 2).reshape(B * 3, 1024, 1024)
    z = _tc_gather(planar, g)
    return z.transpose(0, 2, 3, 1)


# speedup vs baseline: 3.9101x; 1.0773x over previous
import functools

import jax
import jax.numpy as jnp
from jax import lax
from jax.experimental import pallas as pl
from jax.experimental.pallas import tpu as pltpu
from jax.experimental.pallas import tpu_sc as plsc

PS = 64
K = 64
B = 8
G = 16
NP = G * G
ROWS_PER_PATCH = PS
ROW_W = PS * 3
N_TABLE_ROWS = B * 1024 * G
N_OUT_ROWS = B * K * ROWS_PER_PATCH


def _pool_body(hm_ref, out_ref):
    g = pl.program_id(1)
    x = hm_ref[0]
    colsum = jnp.sum(x, axis=0, keepdims=True)
    r = lax.broadcasted_iota(jnp.int32, (1024, G), 0)
    c = lax.broadcasted_iota(jnp.int32, (1024, G), 1)
    grp = ((r // PS) == c).astype(jnp.float32)
    row = lax.dot_general(colsum, grp, (((1,), (0,)), ((), ())),
                          precision=lax.Precision.HIGHEST,
                          preferred_element_type=jnp.float32)
    out_ref[0, pl.ds(g, 1), :] = row * (1.0 / (PS * PS))


def _pool(hm):
    return pl.pallas_call(
        _pool_body,
        grid=(B, G),
        in_specs=[pl.BlockSpec((1, PS, 1024), lambda b, g: (b, g, 0))],
        out_specs=pl.BlockSpec((1, G, G), lambda b, g: (b, 0, 0)),
        out_shape=jax.ShapeDtypeStruct((B, G, G), jnp.float32),
    )(hm)


def _col_of(row_vec, n):
    i = lax.broadcasted_iota(jnp.int32, (n, n), 0)
    j = lax.broadcasted_iota(jnp.int32, (n, n), 1)
    diag = (i == j).astype(row_vec.dtype)
    return jnp.sum(diag * row_vec, axis=1, keepdims=True)


def _topk_body(avg_ref, out_ref):
    b = pl.program_id(0)
    val = avg_ref[0]
    val_col = _col_of(val, NP)
    i_lane = lax.broadcasted_iota(jnp.int32, (NP, NP), 1)
    j_sub = lax.broadcasted_iota(jnp.int32, (NP, NP), 0)
    beats = (val_col > val) | ((val_col == val) & (j_sub < i_lane))
    rank = jnp.sum(beats.astype(jnp.float32), axis=0, keepdims=True)
    rank_col = _col_of(rank, NP)
    p = lax.broadcasted_iota(jnp.int32, (1, K), 1).astype(jnp.float32)
    onehot = (rank_col == p).astype(jnp.float32)
    i_col = lax.broadcasted_iota(jnp.int32, (NP, 1), 0).astype(jnp.float32)
    g_row = jnp.sum(onehot * i_col, axis=0, keepdims=True)
    g_col = _col_of(g_row, K).astype(jnp.int32)
    gh = g_col >> 4
    gw = g_col & 15
    base = b * (1024 * G) + gh * (PS * G) + gw
    step = lax.broadcasted_iota(jnp.int32, (1, ROWS_PER_PATCH), 1) * G
    out_ref[:] = base + step


def _topk_rows(avg):
    return pl.pallas_call(
        _topk_body,
        grid=(B,),
        in_specs=[pl.BlockSpec((1, 1, NP), lambda b: (b, 0, 0))],
        out_specs=pl.BlockSpec((K, ROWS_PER_PATCH), lambda b: (b, 0)),
        out_shape=jax.ShapeDtypeStruct((B * K, ROWS_PER_PATCH), jnp.int32),
    )(avg)


def _sc_pool_topk_body(hm_ref, g_ref, stripe_v, avg16_v, avg2d_v, gsel_v,
                       sh_ref):
    c = lax.axis_index("c")
    s = lax.axis_index("s")
    b_local = s >> 2
    b = c * 4 + b_local

    if True:
        for k in range(4):
            gh = (s & 3) * 4 + k
            row0 = b * 1024 + gh * PS
            pltpu.sync_copy(hm_ref.at[pl.ds(row0, PS)], stripe_v)
            lane16 = lax.broadcasted_iota(jnp.int32, (16,), 0)
            avg_vec = jnp.zeros((16,), jnp.float32)
            for cb in range(4):
                def body(r, acc):
                    return tuple(
                        acc[v] + stripe_v[r, pl.ds(cb * 256 + v * 16, 16)]
                        for v in range(16))
                acc = lax.fori_loop(
                    0, PS, body,
                    tuple(jnp.zeros((16,), jnp.float32) for _ in range(16)))
                for q in range(4):
                    tot = acc[4 * q] + acc[4 * q + 1] + acc[4 * q + 2] \
                        + acc[4 * q + 3]
                    sc = jnp.broadcast_to(jnp.sum(tot), (16,))
                    avg_vec = jnp.where(lane16 == 4 * cb + q, sc, avg_vec)
            avg16_v[...] = avg_vec
            pltpu.sync_copy(avg16_v, sh_ref.at[b_local, gh])
        plsc.subcore_barrier()

        @pl.when(s < 4)
        def _topk():
            pltpu.sync_copy(sh_ref.at[s], avg2d_v)
            lane = lax.broadcasted_iota(jnp.int32, (16,), 0)

            vis = [avg2d_v[i, :] for i in range(16)]

            def body(j, ranks):
                vj = avg2d_v[j, :]
                new = list(ranks)
                for l in range(16):
                    val = jnp.broadcast_to(vj[l], (16,))
                    jl = j * 16 + l
                    for i in range(16):
                        ids_i = i * 16 + lane
                        beats = (val > vis[i]) | ((val == vis[i])
                                                  & (jl < ids_i))
                        new[i] = new[i] + jnp.where(beats, 1, 0)
                return tuple(new)
            ranks = lax.fori_loop(
                0, G, body,
                tuple(jnp.zeros((16,), jnp.int32) for _ in range(16)))
            for i in range(16):
                plsc.store_scatter(gsel_v, [ranks[i]], i * 16 + lane,
                                   mask=ranks[i] < K)
            cb2 = c * 4 + s
            pltpu.sync_copy(gsel_v, g_ref.at[pl.ds(cb2 * K, K)])


def _sc_pool_topk(hm_rows):
    mesh = plsc.VectorSubcoreMesh(core_axis_name="c", subcore_axis_name="s")
    return pl.kernel(
        _sc_pool_topk_body,
        out_type=jax.ShapeDtypeStruct((B * K,), jnp.int32),
        mesh=mesh,
        compiler_params=pltpu.CompilerParams(use_tc_tiling_on_sc=False,
                                             needs_layout_passes=False),
        scratch_types=[
            pltpu.VMEM((PS, 1024), jnp.float32),
            pltpu.VMEM((G,), jnp.float32),
            pltpu.VMEM((G, G), jnp.float32),
            pltpu.VMEM((K,), jnp.int32),
            pltpu.VMEM_SHARED((4, G, G), jnp.float32),
        ],
    )(hm_rows)


CHUNK = 128


def _gather_body(num_cores, rows_per_worker, table_hbm, idx_hbm, out_hbm,
                 idx_v, buf0, buf1, sem0, sem1):
    n_chunks = rows_per_worker // CHUNK
    wid = lax.axis_index("s") * num_cores + lax.axis_index("c")
    pltpu.sync_copy(idx_hbm.at[wid], idx_v)
    bufs = (buf0, buf1)
    sems = (sem0, sem1)
    cps = [None, None]
    cps[0] = pltpu.async_copy(table_hbm.at[idx_v.at[0]], bufs[0], sems[0])
    for c in range(n_chunks):
        nxt = (c + 1) % 2
        if c + 1 < n_chunks:
            cps[nxt] = pltpu.async_copy(
                table_hbm.at[idx_v.at[c + 1]], bufs[nxt], sems[nxt])
        cps[c % 2].wait()
        pltpu.sync_copy(
            bufs[c % 2],
            out_hbm.at[pl.ds(wid * rows_per_worker + c * CHUNK, CHUNK)])


def _gather(table, idx_rows):
    info = plsc.get_sparse_core_info()
    nw = info.num_cores * info.num_subcores
    rows_per_worker = N_OUT_ROWS // nw
    idx3 = idx_rows.reshape(nw, rows_per_worker // CHUNK, CHUNK)
    mesh = plsc.VectorSubcoreMesh(core_axis_name="c", subcore_axis_name="s")
    body = functools.partial(_gather_body, info.num_cores, rows_per_worker)
    return pl.kernel(
        body,
        out_type=jax.ShapeDtypeStruct((N_OUT_ROWS, ROW_W), jnp.float32),
        mesh=mesh,
        compiler_params=pltpu.CompilerParams(use_tc_tiling_on_sc=False),
        scratch_types=[
            pltpu.VMEM((rows_per_worker // CHUNK, CHUNK), jnp.int32),
            pltpu.VMEM((CHUNK, ROW_W), jnp.float32),
            pltpu.VMEM((CHUNK, ROW_W), jnp.float32),
            pltpu.SemaphoreType.DMA,
            pltpu.SemaphoreType.DMA,
        ],
    )(table, idx3)


PPS = 32


def _tc_gather_body(idx_ref, *refs):
    n = pl.program_id(0)
    img_refs, out_ref = refs[:PPS], refs[PPS]
    for j in range(PPS):
        parity = idx_ref[n * PPS + j] & 1
        x = img_refs[j][...]
        out_ref[j] = jnp.where(parity == 0, x[:, :, :PS], x[:, :, PS:])


def _tc_gather(planar, idx):
    def img_map(j):
        def m(n, idx_ref):
            g = idx_ref[n * PPS + j]
            return ((n * PPS + j) // K, g >> 4, (g & 15) >> 1)
        return m

    grid_spec = pltpu.PrefetchScalarGridSpec(
        num_scalar_prefetch=1,
        grid=(B * K // PPS,),
        in_specs=[pl.BlockSpec((3, PS, 2 * PS), img_map(j))
                  for j in range(PPS)],
        out_specs=pl.BlockSpec((PPS, 3, PS, PS),
                               lambda n, idx_ref: (n, 0, 0, 0)),
    )
    return pl.pallas_call(
        _tc_gather_body,
        grid_spec=grid_spec,
        out_shape=jax.ShapeDtypeStruct((B * K, 3, PS, PS), jnp.float32),
    )(idx, *([planar] * PPS))


def kernel(heatmap, image):
    hm_rows = heatmap.reshape(B * 1024, 1024)
    g = _sc_pool_topk(hm_rows)
    planar = image.transpose(0, 3, 1, 2).reshape(B * 3, 1024, 1024)
    z = _tc_gather(planar, g)
    return z.transpose(0, 2, 3, 1)
